# Initial kernel scaffold; baseline (speedup 1.0000x reference)
import jax
import jax.numpy as jnp
from jax.experimental import pallas as pl


def kernel(x, edge_index, W, b):
    raise NotImplementedError("write your pallas kernel here")


# speedup vs baseline: 11.4216x; 11.4216x over previous
import functools

import jax
import jax.numpy as jnp
from jax import lax
from jax.experimental import pallas as pl
from jax.experimental.pallas import tpu as pltpu
from jax.experimental.pallas import tpu_sc as plsc

N = 10000
CH = 128
NCORE = 2
NSUB = 16
NP = 10112
SPR = NP // NSUB
CK = 128
K = 80
CAP = NCORE * NSUB * K * CK
DUMP = N

_mesh = plsc.VectorSubcoreMesh(core_axis_name="c", subcore_axis_name="s")


@functools.partial(
    pl.kernel,
    out_type=jax.ShapeDtypeStruct((NCORE, NP, CH), jnp.float32),
    mesh=_mesh,
    scratch_types=[
        pltpu.VMEM((K, CK), jnp.int32),
        pltpu.VMEM((CK, CH), jnp.float32),
        pltpu.VMEM_SHARED((NP, CH), jnp.float32),
    ],
)
def _deg_kernel(col_hbm, ones_hbm, zeros_hbm, out_hbm, colv, onesv, deg_sh):
    cid = lax.axis_index("c")
    sid = lax.axis_index("s")
    base = sid * SPR
    pltpu.sync_copy(zeros_hbm.at[pl.ds(base, SPR)], deg_sh.at[pl.ds(base, SPR)])
    pltpu.sync_copy(col_hbm.at[cid, sid], colv)
    pltpu.sync_copy(ones_hbm, onesv)
    plsc.subcore_barrier()

    def body(k, carry):
        pltpu.sync_copy(onesv, deg_sh.at[colv.at[k]], add=True)
        return carry

    lax.fori_loop(0, K, body, 0)
    plsc.subcore_barrier()
    pltpu.sync_copy(deg_sh.at[pl.ds(base, SPR)], out_hbm.at[cid, pl.ds(base, SPR)])


@functools.partial(
    pl.kernel,
    out_type=jax.ShapeDtypeStruct((NCORE, NP, CH), jnp.float32),
    mesh=_mesh,
    scratch_types=[
        pltpu.VMEM((K, CK), jnp.int32),
        pltpu.VMEM((K, CK), jnp.int32),
        pltpu.VMEM((CK, CH), jnp.float32),
        pltpu.VMEM_SHARED((NP, CH), jnp.float32),
        pltpu.SemaphoreType.DMA,
    ],
)
def _scat_kernel(row_hbm, col_hbm, g_hbm, zeros_hbm, out_hbm,
                 rowv, colv, buf, acc_sh, sem):
    cid = lax.axis_index("c")
    sid = lax.axis_index("s")
    base = sid * SPR
    pltpu.sync_copy(zeros_hbm.at[pl.ds(base, SPR)], acc_sh.at[pl.ds(base, SPR)])
    pltpu.sync_copy(row_hbm.at[cid, sid], rowv)
    pltpu.sync_copy(col_hbm.at[cid, sid], colv)
    plsc.subcore_barrier()

    def body(k, carry):
        pltpu.async_copy(g_hbm.at[rowv.at[k]], buf, sem).wait()
        pltpu.sync_copy(buf, acc_sh.at[colv.at[k]], add=True)
        return carry

    lax.fori_loop(0, K, body, 0)
    plsc.subcore_barrier()
    pltpu.sync_copy(acc_sh.at[pl.ds(base, SPR)], out_hbm.at[cid, pl.ds(base, SPR)])


def _tc_transform(x, W, dp0, dp1):
    def body(x_ref, w_ref, d0_ref, d1_ref, g_ref, dinv_ref):
        deg = d0_ref[...] + d1_ref[...] + 1.0
        dinv = lax.rsqrt(deg)
        h = jnp.dot(x_ref[...], w_ref[...], preferred_element_type=jnp.float32)
        g_ref[...] = h * dinv
        dinv_ref[...] = dinv

    return pl.pallas_call(
        body,
        out_shape=(
            jax.ShapeDtypeStruct((N, CH), jnp.float32),
            jax.ShapeDtypeStruct((N, 1), jnp.float32),
        ),
    )(x, W, dp0, dp1)


def _tc_combine(acc, g, dinv, b2):
    def body(a_ref, g_ref, di_ref, b_ref, o_ref):
        s = a_ref[0, :N, :] + a_ref[1, :N, :] + g_ref[...]
        o_ref[...] = s * di_ref[...] + b_ref[...]

    return pl.pallas_call(
        body,
        out_shape=jax.ShapeDtypeStruct((N, CH), jnp.float32),
    )(acc, g, dinv, b2)


def kernel(x, edge_index, W, b):
    row = edge_index[0].astype(jnp.int32)
    col = edge_index[1].astype(jnp.int32)
    pad = CAP - row.shape[0]
    row_p = jnp.concatenate([row, jnp.zeros((pad,), jnp.int32)])
    col_p = jnp.concatenate([col, jnp.full((pad,), DUMP, jnp.int32)])
    row_p = row_p.reshape(NCORE, NSUB, K, CK)
    col_p = col_p.reshape(NCORE, NSUB, K, CK)

    ones_rows = jnp.ones((CK, CH), jnp.float32)
    zbig = jnp.zeros((NP, CH), jnp.float32)

    deg_part = _deg_kernel(col_p, ones_rows, zbig)
    dp0 = deg_part[0, :N, 0:1]
    dp1 = deg_part[1, :N, 0:1]
    g, dinv = _tc_transform(x, W, dp0, dp1)
    acc = _scat_kernel(row_p, col_p, g, zbig)
    return _tc_combine(acc, g, dinv, b.reshape(1, CH))


# speedup vs baseline: 12.4042x; 1.0860x over previous
import functools

import jax
import jax.numpy as jnp
from jax import lax
from jax.experimental import pallas as pl
from jax.experimental.pallas import tpu as pltpu
from jax.experimental.pallas import tpu_sc as plsc

N = 10000
CH = 128
NCORE = 2
NSUB = 16
NP = 10112
SPR = NP // NSUB
CK = 128
K = 80
CAP = NCORE * NSUB * K * CK
DUMP = N

_mesh = plsc.VectorSubcoreMesh(core_axis_name="c", subcore_axis_name="s")


@functools.partial(
    pl.kernel,
    out_type=jax.ShapeDtypeStruct((NCORE, NP, CH), jnp.float32),
    mesh=_mesh,
    scratch_types=[
        pltpu.VMEM((K, CK), jnp.int32),
        pltpu.VMEM((CK, CH), jnp.float32),
        pltpu.VMEM_SHARED((NP, CH), jnp.float32),
    ],
)
def _deg_kernel(col_hbm, ones_hbm, zeros_hbm, out_hbm, colv, onesv, deg_sh):
    cid = lax.axis_index("c")
    sid = lax.axis_index("s")
    base = sid * SPR
    pltpu.sync_copy(zeros_hbm.at[pl.ds(base, SPR)], deg_sh.at[pl.ds(base, SPR)])
    pltpu.sync_copy(col_hbm.at[cid, sid], colv)
    pltpu.sync_copy(ones_hbm, onesv)
    plsc.subcore_barrier()

    def body(k, carry):
        pltpu.sync_copy(onesv, deg_sh.at[colv.at[k]], add=True)
        return carry

    lax.fori_loop(0, K, body, 0)
    plsc.subcore_barrier()
    pltpu.sync_copy(deg_sh.at[pl.ds(base, SPR)], out_hbm.at[cid, pl.ds(base, SPR)])


@functools.partial(
    pl.kernel,
    out_type=jax.ShapeDtypeStruct((NCORE, NP, CH), jnp.float32),
    mesh=_mesh,
    scratch_types=[
        pltpu.VMEM((K // 2, CK), jnp.int32),
        pltpu.VMEM((K // 2, CK), jnp.int32),
        pltpu.VMEM((CK, CH), jnp.float32),
        pltpu.VMEM((CK, CH), jnp.float32),
        pltpu.VMEM_SHARED((NP, CH), jnp.float32),
        pltpu.SemaphoreType.DMA,
        pltpu.SemaphoreType.DMA,
    ],
)
def _scat_kernel(row_hbm, col_hbm, g_hbm, zeros_hbm, out_hbm,
                 rowv, colv, buf0, buf1, acc_sh, sem0, sem1):
    cid = lax.axis_index("c")
    sid = lax.axis_index("s")
    base = sid * SPR
    K2 = K // 2
    pltpu.sync_copy(zeros_hbm.at[pl.ds(base, SPR)], acc_sh.at[pl.ds(base, SPR)])
    plsc.subcore_barrier()

    for half in range(2):
        pltpu.sync_copy(row_hbm.at[cid, sid, pl.ds(half * K2, K2)], rowv)
        pltpu.sync_copy(col_hbm.at[cid, sid, pl.ds(half * K2, K2)], colv)
        pltpu.async_copy(g_hbm.at[rowv.at[0]], buf0, sem0)
        pltpu.async_copy(g_hbm.at[rowv.at[1]], buf1, sem1)

        def body(j, carry):
            k0 = 2 * j
            pltpu.make_async_copy(g_hbm.at[rowv.at[k0]], buf0, sem0).wait()
            pltpu.sync_copy(buf0, acc_sh.at[colv.at[k0]], add=True)

            @pl.when(k0 + 2 < K2)
            def _():
                pltpu.async_copy(g_hbm.at[rowv.at[k0 + 2]], buf0, sem0)

            pltpu.make_async_copy(g_hbm.at[rowv.at[k0 + 1]], buf1, sem1).wait()
            pltpu.sync_copy(buf1, acc_sh.at[colv.at[k0 + 1]], add=True)

            @pl.when(k0 + 3 < K2)
            def _():
                pltpu.async_copy(g_hbm.at[rowv.at[k0 + 3]], buf1, sem1)

            return carry

        lax.fori_loop(0, K2 // 2, body, 0)
    plsc.subcore_barrier()
    pltpu.sync_copy(acc_sh.at[pl.ds(base, SPR)], out_hbm.at[cid, pl.ds(base, SPR)])


def _tc_transform(x, W, dp0, dp1):
    def body(x_ref, w_ref, d0_ref, d1_ref, g_ref, dinv_ref):
        deg = d0_ref[...] + d1_ref[...] + 1.0
        dinv = lax.rsqrt(deg)
        h = jnp.dot(x_ref[...], w_ref[...], preferred_element_type=jnp.float32)
        g_ref[...] = h * dinv
        dinv_ref[...] = dinv

    return pl.pallas_call(
        body,
        out_shape=(
            jax.ShapeDtypeStruct((N, CH), jnp.float32),
            jax.ShapeDtypeStruct((N, 1), jnp.float32),
        ),
    )(x, W, dp0, dp1)


def _tc_combine(acc, g, dinv, b2):
    def body(a_ref, g_ref, di_ref, b_ref, o_ref):
        s = a_ref[0, :N, :] + a_ref[1, :N, :] + g_ref[...]
        o_ref[...] = s * di_ref[...] + b_ref[...]

    return pl.pallas_call(
        body,
        out_shape=jax.ShapeDtypeStruct((N, CH), jnp.float32),
    )(acc, g, dinv, b2)


def kernel(x, edge_index, W, b):
    row = edge_index[0].astype(jnp.int32)
    col = edge_index[1].astype(jnp.int32)
    pad = CAP - row.shape[0]
    row_p = jnp.concatenate([row, jnp.zeros((pad,), jnp.int32)])
    col_p = jnp.concatenate([col, jnp.full((pad,), DUMP, jnp.int32)])
    row_p = row_p.reshape(NCORE, NSUB, K, CK)
    col_p = col_p.reshape(NCORE, NSUB, K, CK)

    ones_rows = jnp.ones((CK, CH), jnp.float32)
    zbig = jnp.zeros((NP, CH), jnp.float32)

    deg_part = _deg_kernel(col_p, ones_rows, zbig)
    dp0 = deg_part[0, :N, 0:1]
    dp1 = deg_part[1, :N, 0:1]
    g, dinv = _tc_transform(x, W, dp0, dp1)
    acc = _scat_kernel(row_p, col_p, g, zbig)
    return _tc_combine(acc, g, dinv, b.reshape(1, CH))
